# Initial kernel scaffold; baseline (speedup 1.0000x reference)
import jax
import jax.numpy as jnp
from jax.experimental import pallas as pl


def kernel(x, W):
    raise NotImplementedError("write your pallas kernel here")


# speedup vs baseline: 1.5690x; 1.5690x over previous
import functools

import jax
import jax.numpy as jnp
from jax import lax
from jax.experimental import pallas as pl
from jax.experimental.pallas import tpu as pltpu
from jax.experimental.pallas import tpu_sc as plsc

_NUM_CORES = 2
_NUM_SUBCORES = 16
_NUM_WORKERS = _NUM_CORES * _NUM_SUBCORES


@functools.lru_cache(maxsize=None)
def _build(B, D, chunk):
    b_per_w = B // _NUM_WORKERS
    n_chunks = b_per_w // chunk
    mesh = plsc.VectorSubcoreMesh(
        core_axis_name="c", subcore_axis_name="s", num_cores=_NUM_CORES
    )

    @functools.partial(
        pl.kernel,
        mesh=mesh,
        out_type=jax.ShapeDtypeStruct((B, D), jnp.float32),
        scratch_types=[
            pltpu.VMEM((b_per_w,), jnp.int32),
            pltpu.VMEM((2, chunk, D), jnp.float32),
            pltpu.SemaphoreType.DMA,
            pltpu.SemaphoreType.DMA,
            pltpu.SemaphoreType.DMA,
            pltpu.SemaphoreType.DMA,
        ],
        compiler_params=pltpu.CompilerParams(use_tc_tiling_on_sc=False),
    )
    def embed(table_hbm, idx_hbm, out_hbm, idx_v, rows_v, g0, g1, s0, s1):
        wid = lax.axis_index("s") * _NUM_CORES + lax.axis_index("c")
        base = wid * b_per_w
        pltpu.sync_copy(idx_hbm.at[pl.ds(base, b_per_w)], idx_v)

        gsem = (g0, g1)
        ssem = (s0, s1)

        def gather(j):
            return pltpu.async_copy(
                table_hbm.at[idx_v.at[pl.ds(j * chunk, chunk)]],
                rows_v.at[j % 2],
                gsem[j % 2],
            )

        def store(j):
            return pltpu.async_copy(
                rows_v.at[j % 2],
                out_hbm.at[pl.ds(base + j * chunk, chunk)],
                ssem[j % 2],
            )

        gathers = [None] * n_chunks
        stores = [None] * n_chunks
        gathers[0] = gather(0)
        for j in range(n_chunks):
            gathers[j].wait()
            if j + 1 < n_chunks:
                if j - 1 >= 0:
                    stores[j - 1].wait()
                gathers[j + 1] = gather(j + 1)
            stores[j] = store(j)
        if n_chunks >= 2:
            stores[n_chunks - 2].wait()
        stores[n_chunks - 1].wait()

    return embed


def kernel(x, W):
    B = x.shape[0] * x.shape[1]
    D = W.shape[1]
    idx = x.reshape(-1).astype(jnp.int32)
    out = _build(B, D, 1664)(W, idx)
    return out.reshape(x.shape + (D,))
